# baseline (speedup 1.0000x reference)
import reference as _r


def kernel(*args):
    return _r.reference(*args)


# speedup vs baseline: 4.8633x; 4.8633x over previous
import functools

import jax
import jax.numpy as jnp
from jax import lax
from jax.experimental import pallas as pl
from jax.experimental.pallas import tpu as pltpu

_STAGES = [(64, 3, 1), (128, 4, 2), (256, 6, 2), (512, 3, 2)]


def _rup(x, m):
    return (x + m - 1) // m * m


def _mm_body(a_ref, b_ref, *rest, has_bias, has_res, act):
    k = pl.program_id(2)

    @pl.when(k == 0)
    def _():
        rest[-1][...] = jnp.zeros_like(rest[-1])

    rest[-1][...] += jnp.dot(a_ref[...], b_ref[...],
                             preferred_element_type=jnp.float32)

    @pl.when(k == pl.num_programs(2) - 1)
    def _():
        r = rest[-1][...]
        if has_bias:
            r = r + rest[0][...]
        if has_res:
            r = r + rest[has_bias][...].astype(jnp.float32)
        if act == "relu":
            r = jnp.maximum(r, 0.0)
        o_ref = rest[has_bias + has_res]
        o_ref[...] = r.astype(o_ref.dtype)


def _mm(a, b, bias=None, res=None, act=None, out_dtype=jnp.bfloat16,
        tm=512, tn=512, tk=1024):
    M, K = a.shape
    _, N = b.shape
    tm = min(tm, _rup(M, 8))
    tn = min(tn, _rup(N, 128))
    tk = min(tk, _rup(K, 128))
    Mp, Np, Kp = _rup(M, tm), _rup(N, tn), _rup(K, tk)

    a = a.astype(jnp.bfloat16)
    b = b.astype(jnp.bfloat16)
    if (Mp, Kp) != (M, K):
        a = jnp.pad(a, ((0, Mp - M), (0, Kp - K)))
    if (Kp, Np) != (K, N):
        b = jnp.pad(b, ((0, Kp - K), (0, Np - N)))

    ops = [a, b]
    specs = [pl.BlockSpec((tm, tk), lambda i, j, k: (i, k)),
             pl.BlockSpec((tk, tn), lambda i, j, k: (k, j))]
    if bias is not None:
        bz = bias.astype(jnp.float32).reshape(1, N)
        if Np != N:
            bz = jnp.pad(bz, ((0, 0), (0, Np - N)))
        ops.append(bz)
        specs.append(pl.BlockSpec((1, tn), lambda i, j, k: (0, j)))
    if res is not None:
        r = res
        if (Mp, Np) != r.shape:
            r = jnp.pad(r, ((0, Mp - M), (0, Np - N)))
        ops.append(r)
        specs.append(pl.BlockSpec((tm, tn), lambda i, j, k: (i, j)))

    out = pl.pallas_call(
        functools.partial(_mm_body, has_bias=bias is not None,
                          has_res=res is not None, act=act),
        out_shape=jax.ShapeDtypeStruct((Mp, Np), out_dtype),
        grid=(Mp // tm, Np // tn, Kp // tk),
        in_specs=specs,
        out_specs=pl.BlockSpec((tm, tn), lambda i, j, k: (i, j)),
        scratch_shapes=[pltpu.VMEM((tm, tn), jnp.float32)],
        compiler_params=pltpu.CompilerParams(
            dimension_semantics=("parallel", "parallel", "arbitrary")),
    )(*ops)
    if (Mp, Np) != (M, N):
        out = out[:M, :N]
    return out


def _stem_body(pe_ref, po_ref, w_ref, o_ref, s_ref):
    s_ref[...] = jnp.zeros_like(s_ref)

    def quad(j, carry):
        for ci in range(3):
            c0 = pe_ref[0, ci, j].astype(jnp.bfloat16)
            c1 = pe_ref[0, ci, j + 1].astype(jnp.bfloat16)
            d0 = po_ref[0, ci, j].astype(jnp.bfloat16)
            d1 = po_ref[0, ci, j + 1].astype(jnp.bfloat16)
            for di in range(4):
                for kh in range(7):
                    r8 = 2 * di + kh
                    ce = c0[r8] if r8 < 8 else c1[r8 - 8]
                    co_ = d0[r8] if r8 < 8 else d1[r8 - 8]
                    for kw in range(7):
                        r = kh * 21 + kw * 3 + ci
                        q = kw // 2 if kw % 2 == 0 else (kw - 1) // 2
                        src = ce if kw % 2 == 0 else co_
                        s_ref[r, di * 128:di * 128 + 112] = src[q:q + 112]
        acc = lax.dot_general(s_ref[...], w_ref[...],
                              (((0,), (0,)), ((), ())),
                              preferred_element_type=jnp.float32)
        y = jnp.maximum(acc, 0.0).astype(o_ref.dtype)
        for di in range(4):
            o_ref[0, pl.ds(4 * j + di, 1)] = y[None, di * 128:di * 128 + 112]
        return carry

    lax.fori_loop(0, 28, quad, 0)


def _stem(x, stem_w):
    xr = x.reshape(16, 3, 224, 224).astype(jnp.bfloat16).astype(jnp.float32)
    xp = jnp.pad(xr, ((0, 0), (0, 0), (3, 13), (3, 3)))
    pe = xp[..., 0::2].reshape(16, 3, 30, 8, 115)
    po = xp[..., 1::2].reshape(16, 3, 30, 8, 115)
    w = jnp.pad(stem_w, ((0, 13), (0, 0)))

    return pl.pallas_call(
        _stem_body,
        out_shape=jax.ShapeDtypeStruct((16, 112, 112, 128), jnp.bfloat16),
        grid=(16,),
        in_specs=[pl.BlockSpec((1, 3, 30, 8, 115), lambda n: (n, 0, 0, 0, 0)),
                  pl.BlockSpec((1, 3, 30, 8, 115), lambda n: (n, 0, 0, 0, 0)),
                  pl.BlockSpec((160, 128), lambda n: (0, 0))],
        out_specs=pl.BlockSpec((1, 112, 112, 128), lambda n: (n, 0, 0, 0)),
        scratch_shapes=[pltpu.VMEM((160, 512), jnp.bfloat16)],
        compiler_params=pltpu.CompilerParams(
            dimension_semantics=("parallel",)),
    )(pe, po, w)


def _maxpool(x):
    return lax.reduce_window(
        x, jnp.asarray(-jnp.inf, dtype=x.dtype), lax.max,
        (1, 3, 3, 1), (1, 2, 2, 1), [(0, 0), (1, 1), (1, 1), (0, 0)])


def _gconv_body(c_ref, w_ref, o_ref, acc_ref):
    t = pl.program_id(2)

    @pl.when(t == 0)
    def _():
        acc_ref[...] = jnp.zeros_like(acc_ref)

    acc_ref[...] += jnp.dot(c_ref[...], w_ref[...],
                            preferred_element_type=jnp.float32)

    @pl.when(t == pl.num_programs(2) - 1)
    def _():
        o_ref[...] = jnp.maximum(acc_ref[...], 0.0).astype(o_ref.dtype)


def _gconv(x, wb, stride):
    N, H, W, C = x.shape
    nb = C // 128
    xp = jnp.pad(x, ((0, 0), (1, 1), (1, 1), (0, 0)))
    Ho = (H + 2 - 3) // stride + 1
    Wo = (W + 2 - 3) // stride + 1
    views = [xp[:, i:i + stride * Ho:stride, j:j + stride * Wo:stride, :]
             for i in range(3) for j in range(3)]
    cols = jnp.concatenate(views, axis=-1).reshape(N * Ho * Wo, 9 * C)
    M = cols.shape[0]
    tm = min(512, _rup(M, 8))
    Mp = _rup(M, tm)
    if Mp != M:
        cols = jnp.pad(cols, ((0, Mp - M), (0, 0)))

    out = pl.pallas_call(
        _gconv_body,
        out_shape=jax.ShapeDtypeStruct((Mp, C), jnp.bfloat16),
        grid=(Mp // tm, nb, 9),
        in_specs=[pl.BlockSpec((tm, 128), lambda i, b, t: (i, t * nb + b)),
                  pl.BlockSpec((128, 128), lambda i, b, t: (b * 9 + t, 0))],
        out_specs=pl.BlockSpec((tm, 128), lambda i, b, t: (i, b)),
        scratch_shapes=[pltpu.VMEM((tm, 128), jnp.float32)],
        compiler_params=pltpu.CompilerParams(
            dimension_semantics=("parallel", "parallel", "arbitrary")),
    )(cols, wb)
    return out[:M].reshape(N, Ho, Wo, C)


def _block(x, w1, w2, w3, wd, stride):
    N, H, W, C = x.shape
    a = _mm(x.reshape(N * H * W, C), w1, act="relu")
    a = a.reshape(N, H, W, -1)
    g = _gconv(a, w2, stride)
    _, Ho, Wo, Cw = g.shape
    if wd is not None:
        xs = x[:, ::stride, ::stride, :] if stride != 1 else x
        ident = _mm(xs.reshape(N * Ho * Wo, C), wd)
    else:
        ident = x.reshape(N * Ho * Wo, C)
    y = _mm(g.reshape(N * Ho * Wo, Cw), w3, res=ident, act="relu")
    return y.reshape(N, Ho, Wo, -1)


def _pool_body(x_ref, o_ref):
    o_ref[...] = jnp.mean(x_ref[...].astype(jnp.float32), axis=1)


def _avgpool(fmap):
    N, H, W, C = fmap.shape
    return pl.pallas_call(
        _pool_body,
        out_shape=jax.ShapeDtypeStruct((N, C), jnp.float32),
    )(fmap.reshape(N, H * W, C))


def _lstm_body(xp_ref, whh_ref, o_ref, *, hidden):
    H = hidden

    def gates(v):
        return (jax.nn.sigmoid(v[:, 0 * H:1 * H]),
                jax.nn.sigmoid(v[:, 1 * H:2 * H]),
                jnp.tanh(v[:, 2 * H:3 * H]),
                jax.nn.sigmoid(v[:, 3 * H:4 * H]))

    i0, f0, g0, o0 = gates(xp_ref[0])
    c1 = i0 * g0
    h1 = o0 * jnp.tanh(c1)
    o_ref[0] = h1

    v1 = xp_ref[1] + jnp.dot(h1.astype(jnp.bfloat16), whh_ref[...],
                             preferred_element_type=jnp.float32)
    i1, f1, g1, o1 = gates(v1)
    c2 = f1 * c1 + i1 * g1
    h2 = o1 * jnp.tanh(c2)
    o_ref[1] = h2


def _lstm(feat, w_ih_t, w_hh_t):
    T, B, D = feat.shape
    H = w_hh_t.shape[0]
    xproj = _mm(feat.reshape(T * B, D), w_ih_t, out_dtype=jnp.float32,
                tn=1024, tk=2048).reshape(T, B, 4 * H)
    return pl.pallas_call(
        functools.partial(_lstm_body, hidden=H),
        out_shape=jax.ShapeDtypeStruct((T, B, H), jnp.float32),
        compiler_params=pltpu.CompilerParams(
            vmem_limit_bytes=50 * 1024 * 1024),
    )(xproj, w_hh_t)


def kernel(x, stem,
           L0_B0_w1, L0_B0_w2, L0_B0_w3, L0_B0_wd,
           L0_B1_w1, L0_B1_w2, L0_B1_w3,
           L0_B2_w1, L0_B2_w2, L0_B2_w3,
           L1_B0_w1, L1_B0_w2, L1_B0_w3, L1_B0_wd,
           L1_B1_w1, L1_B1_w2, L1_B1_w3,
           L1_B2_w1, L1_B2_w2, L1_B2_w3,
           L1_B3_w1, L1_B3_w2, L1_B3_w3,
           L2_B0_w1, L2_B0_w2, L2_B0_w3, L2_B0_wd,
           L2_B1_w1, L2_B1_w2, L2_B1_w3,
           L2_B2_w1, L2_B2_w2, L2_B2_w3,
           L2_B3_w1, L2_B3_w2, L2_B3_w3,
           L2_B4_w1, L2_B4_w2, L2_B4_w3,
           L2_B5_w1, L2_B5_w2, L2_B5_w3,
           L3_B0_w1, L3_B0_w2, L3_B0_w3, L3_B0_wd,
           L3_B1_w1, L3_B1_w2, L3_B1_w3,
           L3_B2_w1, L3_B2_w2, L3_B2_w3,
           w_ih_t, w_hh_t, fc_w_t, fc_b):
    blocks = [
        [(L0_B0_w1, L0_B0_w2, L0_B0_w3, L0_B0_wd),
         (L0_B1_w1, L0_B1_w2, L0_B1_w3, None),
         (L0_B2_w1, L0_B2_w2, L0_B2_w3, None)],
        [(L1_B0_w1, L1_B0_w2, L1_B0_w3, L1_B0_wd),
         (L1_B1_w1, L1_B1_w2, L1_B1_w3, None),
         (L1_B2_w1, L1_B2_w2, L1_B2_w3, None),
         (L1_B3_w1, L1_B3_w2, L1_B3_w3, None)],
        [(L2_B0_w1, L2_B0_w2, L2_B0_w3, L2_B0_wd),
         (L2_B1_w1, L2_B1_w2, L2_B1_w3, None),
         (L2_B2_w1, L2_B2_w2, L2_B2_w3, None),
         (L2_B3_w1, L2_B3_w2, L2_B3_w3, None),
         (L2_B4_w1, L2_B4_w2, L2_B4_w3, None),
         (L2_B5_w1, L2_B5_w2, L2_B5_w3, None)],
        [(L3_B0_w1, L3_B0_w2, L3_B0_w3, L3_B0_wd),
         (L3_B1_w1, L3_B1_w2, L3_B1_w3, None),
         (L3_B2_w1, L3_B2_w2, L3_B2_w3, None)],
    ]

    v = _maxpool(_stem(x, stem))
    for (planes, _n, stride), stage in zip(_STAGES, blocks):
        for bi, (w1, w2, w3, wd) in enumerate(stage):
            v = _block(v, w1, w2, w3, wd, stride if bi == 0 else 1)

    fmap_nhwc = v
    pooled = _avgpool(fmap_nhwc)
    feat = pooled.reshape(2, 8, 2048)
    h_all = _lstm(feat, w_ih_t, w_hh_t)
    last = h_all[:, -1, :]
    logits = _mm(last, fc_w_t, bias=fc_b, out_dtype=jnp.float32, tk=2048)
    fmap = fmap_nhwc.transpose(0, 3, 1, 2).astype(jnp.float32)
    return fmap, logits


# speedup vs baseline: 8.1991x; 1.6859x over previous
import functools

import jax
import jax.numpy as jnp
from jax import lax
from jax.experimental import pallas as pl
from jax.experimental.pallas import tpu as pltpu

_STAGES = [(64, 3, 1), (128, 4, 2), (256, 6, 2), (512, 3, 2)]


def _rup(x, m):
    return (x + m - 1) // m * m


def _mm_body(a_ref, b_ref, *rest, has_bias, has_res, act):
    k = pl.program_id(2)

    @pl.when(k == 0)
    def _():
        rest[-1][...] = jnp.zeros_like(rest[-1])

    rest[-1][...] += jnp.dot(a_ref[...], b_ref[...],
                             preferred_element_type=jnp.float32)

    @pl.when(k == pl.num_programs(2) - 1)
    def _():
        r = rest[-1][...]
        if has_bias:
            r = r + rest[0][...]
        if has_res:
            r = r + rest[has_bias][...].astype(jnp.float32)
        if act == "relu":
            r = jnp.maximum(r, 0.0)
        o_ref = rest[has_bias + has_res]
        o_ref[...] = r.astype(o_ref.dtype)


def _mm(a, b, bias=None, res=None, act=None, out_dtype=jnp.bfloat16,
        tm=512, tn=512, tk=1024):
    M, K = a.shape
    _, N = b.shape
    tm = min(tm, _rup(M, 8))
    tn = min(tn, _rup(N, 128))
    tk = min(tk, _rup(K, 128))
    Mp, Np, Kp = _rup(M, tm), _rup(N, tn), _rup(K, tk)

    a = a.astype(jnp.bfloat16)
    b = b.astype(jnp.bfloat16)
    if (Mp, Kp) != (M, K):
        a = jnp.pad(a, ((0, Mp - M), (0, Kp - K)))
    if (Kp, Np) != (K, N):
        b = jnp.pad(b, ((0, Kp - K), (0, Np - N)))

    ops = [a, b]
    specs = [pl.BlockSpec((tm, tk), lambda i, j, k: (i, k)),
             pl.BlockSpec((tk, tn), lambda i, j, k: (k, j))]
    if bias is not None:
        bz = bias.astype(jnp.float32).reshape(1, N)
        if Np != N:
            bz = jnp.pad(bz, ((0, 0), (0, Np - N)))
        ops.append(bz)
        specs.append(pl.BlockSpec((1, tn), lambda i, j, k: (0, j)))
    if res is not None:
        r = res
        if (Mp, Np) != r.shape:
            r = jnp.pad(r, ((0, Mp - M), (0, Np - N)))
        ops.append(r)
        specs.append(pl.BlockSpec((tm, tn), lambda i, j, k: (i, j)))

    out = pl.pallas_call(
        functools.partial(_mm_body, has_bias=bias is not None,
                          has_res=res is not None, act=act),
        out_shape=jax.ShapeDtypeStruct((Mp, Np), out_dtype),
        grid=(Mp // tm, Np // tn, Kp // tk),
        in_specs=specs,
        out_specs=pl.BlockSpec((tm, tn), lambda i, j, k: (i, j)),
        scratch_shapes=[pltpu.VMEM((tm, tn), jnp.float32)],
        compiler_params=pltpu.CompilerParams(
            dimension_semantics=("parallel", "parallel", "arbitrary")),
    )(*ops)
    if (Mp, Np) != (M, N):
        out = out[:M, :N]
    return out


def _stem_body(pe_ref, po_ref, w_ref, o_ref, s_ref):
    s_ref[...] = jnp.zeros_like(s_ref)

    def quad(j, carry):
        for ci in range(3):
            c0 = pe_ref[0, ci, j].astype(jnp.bfloat16)
            c1 = pe_ref[0, ci, j + 1].astype(jnp.bfloat16)
            d0 = po_ref[0, ci, j].astype(jnp.bfloat16)
            d1 = po_ref[0, ci, j + 1].astype(jnp.bfloat16)
            for di in range(4):
                for kh in range(7):
                    r8 = 2 * di + kh
                    ce = c0[r8] if r8 < 8 else c1[r8 - 8]
                    co_ = d0[r8] if r8 < 8 else d1[r8 - 8]
                    for kw in range(7):
                        r = kh * 21 + kw * 3 + ci
                        q = kw // 2 if kw % 2 == 0 else (kw - 1) // 2
                        src = ce if kw % 2 == 0 else co_
                        s_ref[r, di * 128:di * 128 + 112] = src[q:q + 112]
        acc = lax.dot_general(s_ref[...], w_ref[...],
                              (((0,), (0,)), ((), ())),
                              preferred_element_type=jnp.float32)
        y = jnp.maximum(acc, 0.0).astype(o_ref.dtype)
        for di in range(4):
            o_ref[0, pl.ds(4 * j + di, 1)] = y[None, di * 128:di * 128 + 112]
        return carry

    lax.fori_loop(0, 28, quad, 0)


def _stem(x, stem_w):
    xr = x.reshape(16, 3, 224, 224).astype(jnp.bfloat16).astype(jnp.float32)
    xp = jnp.pad(xr, ((0, 0), (0, 0), (3, 13), (3, 3)))
    pe = xp[..., 0::2].reshape(16, 3, 30, 8, 115)
    po = xp[..., 1::2].reshape(16, 3, 30, 8, 115)
    w = jnp.pad(stem_w, ((0, 13), (0, 0)))

    return pl.pallas_call(
        _stem_body,
        out_shape=jax.ShapeDtypeStruct((16, 112, 112, 128), jnp.bfloat16),
        grid=(16,),
        in_specs=[pl.BlockSpec((1, 3, 30, 8, 115), lambda n: (n, 0, 0, 0, 0)),
                  pl.BlockSpec((1, 3, 30, 8, 115), lambda n: (n, 0, 0, 0, 0)),
                  pl.BlockSpec((160, 128), lambda n: (0, 0))],
        out_specs=pl.BlockSpec((1, 112, 112, 128), lambda n: (n, 0, 0, 0)),
        scratch_shapes=[pltpu.VMEM((160, 512), jnp.bfloat16)],
        compiler_params=pltpu.CompilerParams(
            dimension_semantics=("parallel",)),
    )(pe, po, w)


def _maxpool(x):
    return lax.reduce_window(
        x, jnp.asarray(-jnp.inf, dtype=x.dtype), lax.max,
        (1, 3, 3, 1), (1, 2, 2, 1), [(0, 0), (1, 1), (1, 1), (0, 0)])


def _gconv_body(c_ref, w_ref, o_ref, acc_ref):
    t = pl.program_id(2)

    @pl.when(t == 0)
    def _():
        acc_ref[...] = jnp.zeros_like(acc_ref)

    acc_ref[...] += jnp.dot(c_ref[...], w_ref[...],
                            preferred_element_type=jnp.float32)

    @pl.when(t == pl.num_programs(2) - 1)
    def _():
        o_ref[...] = jnp.maximum(acc_ref[...], 0.0).astype(o_ref.dtype)


def _gconv(x, wb, stride):
    N, H, W, C = x.shape
    nb = C // 128
    xp = jnp.pad(x, ((0, 0), (1, 1), (1, 1), (0, 0)))
    Ho = (H + 2 - 3) // stride + 1
    Wo = (W + 2 - 3) // stride + 1
    views = [xp[:, i:i + stride * Ho:stride, j:j + stride * Wo:stride, :]
             for i in range(3) for j in range(3)]
    cols = jnp.concatenate(views, axis=-1).reshape(N * Ho * Wo, 9 * C)
    M = cols.shape[0]
    tm = min(512, _rup(M, 8))
    Mp = _rup(M, tm)
    if Mp != M:
        cols = jnp.pad(cols, ((0, Mp - M), (0, 0)))

    out = pl.pallas_call(
        _gconv_body,
        out_shape=jax.ShapeDtypeStruct((Mp, C), jnp.bfloat16),
        grid=(Mp // tm, nb, 9),
        in_specs=[pl.BlockSpec((tm, 128), lambda i, b, t: (i, t * nb + b)),
                  pl.BlockSpec((128, 128), lambda i, b, t: (b * 9 + t, 0))],
        out_specs=pl.BlockSpec((tm, 128), lambda i, b, t: (i, b)),
        scratch_shapes=[pltpu.VMEM((tm, 128), jnp.float32)],
        compiler_params=pltpu.CompilerParams(
            dimension_semantics=("parallel", "parallel", "arbitrary")),
    )(cols, wb)
    return out[:M].reshape(N, Ho, Wo, C)


def _fblock_body(*refs, geom, nb, has_wd, ksplit):
    Hp, Wp, W = geom
    clen = (Hp - 4) * Wp
    x_ref, w1_ref, wb_ref, w3_ref = refs[:4]
    wd_ref = refs[4] if has_wd else None
    o_ref = refs[4 + has_wd]
    a_ref, gacc_ref = refs[5 + has_wd:7 + has_wd]
    t = pl.program_id(1)

    if ksplit:
        aacc_ref = refs[7 + has_wd]

        @pl.when(t == 0)
        def _():
            aacc_ref[...] = jnp.dot(x_ref[:, :1024], w1_ref[:1024],
                                    preferred_element_type=jnp.float32)

        @pl.when(t == 1)
        def _():
            acc = aacc_ref[...] + jnp.dot(x_ref[:, 1024:], w1_ref[1024:],
                                          preferred_element_type=jnp.float32)
            a_ref[...] = jnp.maximum(acc, 0.0).astype(jnp.bfloat16)
    else:
        @pl.when(t == 0)
        def _():
            a = jnp.dot(x_ref[...], w1_ref[...],
                        preferred_element_type=jnp.float32)
            a_ref[...] = jnp.maximum(a, 0.0).astype(jnp.bfloat16)

    k0 = 2 if ksplit else 1
    for ti in range(9):
        @pl.when(t == k0 + ti)
        def _(ti=ti):
            di, dj = ti // 3, ti % 3
            base = (di + 1) * Wp + dj - 1
            for b in range(nb):
                part = jnp.dot(
                    a_ref[base:base + clen, b * 128:b * 128 + 128],
                    wb_ref[(b * 9 + ti) * 128:(b * 9 + ti) * 128 + 128, :],
                    preferred_element_type=jnp.float32)
                if ti == 0:
                    gacc_ref[:, b * 128:b * 128 + 128] = part
                else:
                    gacc_ref[:, b * 128:b * 128 + 128] += part

    @pl.when(t == k0 + 9)
    def _():
        r = lax.broadcasted_iota(jnp.int32, gacc_ref.shape, 0) % Wp
        colmask = (r >= 1) & (r <= W)
        g = jnp.where(colmask, jnp.maximum(gacc_ref[...], 0.0), 0.0)
        g = g.astype(jnp.bfloat16)
        y = jnp.dot(g, w3_ref[...], preferred_element_type=jnp.float32)
        xc = x_ref[2 * Wp:2 * Wp + clen, :]
        if has_wd:
            res = jnp.dot(xc, wd_ref[...], preferred_element_type=jnp.float32)
            res = res.astype(jnp.bfloat16).astype(jnp.float32)
        else:
            res = xc.astype(jnp.float32)
        outc = o_ref.shape[1]
        o_ref[0:2 * Wp, :] = jnp.zeros((2 * Wp, outc), o_ref.dtype)
        o_ref[2 * Wp:2 * Wp + clen, :] = jnp.maximum(y + res, 0.0).astype(o_ref.dtype)
        o_ref[2 * Wp + clen:, :] = jnp.zeros((2 * Wp, outc), o_ref.dtype)


def _fblock(x, w1, w2, w3, wd, geom):
    Hp, Wp, W = geom
    Rf = Hp * Wp
    Cin = x.shape[1]
    width = w1.shape[1]
    outc = w3.shape[1]
    nb = width // 128
    ksplit = Cin > 1024
    nsteps = (2 if ksplit else 1) + 10
    ops = [x, w1, w2, w3] + ([wd] if wd is not None else [])
    specs = [pl.BlockSpec((Rf, Cin), lambda n, t: (n, 0)),
             pl.BlockSpec(w1.shape, lambda n, t: (0, 0)),
             pl.BlockSpec(w2.shape, lambda n, t: (0, 0)),
             pl.BlockSpec(w3.shape, lambda n, t: (0, 0))]
    if wd is not None:
        specs.append(pl.BlockSpec(wd.shape, lambda n, t: (0, 0)))
    return pl.pallas_call(
        functools.partial(_fblock_body, geom=geom, nb=nb,
                          has_wd=wd is not None, ksplit=ksplit),
        out_shape=jax.ShapeDtypeStruct((16 * Rf, outc), jnp.bfloat16),
        grid=(16, nsteps),
        in_specs=specs,
        out_specs=pl.BlockSpec((Rf, outc), lambda n, t: (n, 0)),
        scratch_shapes=[pltpu.VMEM((Rf, width), jnp.bfloat16),
                        pltpu.VMEM(((Hp - 4) * Wp, width), jnp.float32)]
        + ([pltpu.VMEM((Rf, width), jnp.float32)] if ksplit else []),
        compiler_params=pltpu.CompilerParams(
            dimension_semantics=("parallel", "arbitrary"),
            vmem_limit_bytes=100 * 1024 * 1024),
    )(*ops)


def _to_flat(v, geom):
    N, H, W, C = v.shape
    Hp, Wp, _ = geom
    vp = jnp.pad(v, ((0, 0), (2, Hp - H - 2), (1, Wp - W - 1), (0, 0)))
    return vp.reshape(N * Hp * Wp, C)


def _from_flat(v, geom, H, W):
    Hp, Wp, _ = geom
    return v.reshape(16, Hp, Wp, -1)[:, 2:2 + H, 1:1 + W, :]


def _block(x, w1, w2, w3, wd, stride):
    N, H, W, C = x.shape
    a = _mm(x.reshape(N * H * W, C), w1, act="relu")
    a = a.reshape(N, H, W, -1)
    g = _gconv(a, w2, stride)
    _, Ho, Wo, Cw = g.shape
    if wd is not None:
        xs = x[:, ::stride, ::stride, :] if stride != 1 else x
        ident = _mm(xs.reshape(N * Ho * Wo, C), wd)
    else:
        ident = x.reshape(N * Ho * Wo, C)
    y = _mm(g.reshape(N * Ho * Wo, Cw), w3, res=ident, act="relu")
    return y.reshape(N, Ho, Wo, -1)


def _pool_body(x_ref, o_ref):
    o_ref[...] = jnp.mean(x_ref[...].astype(jnp.float32), axis=1)


def _avgpool(fmap):
    N, H, W, C = fmap.shape
    return pl.pallas_call(
        _pool_body,
        out_shape=jax.ShapeDtypeStruct((N, C), jnp.float32),
    )(fmap.reshape(N, H * W, C))


def _lstm_body(xp_ref, whh_ref, o_ref, *, hidden):
    H = hidden

    def gates(v):
        return (jax.nn.sigmoid(v[:, 0 * H:1 * H]),
                jax.nn.sigmoid(v[:, 1 * H:2 * H]),
                jnp.tanh(v[:, 2 * H:3 * H]),
                jax.nn.sigmoid(v[:, 3 * H:4 * H]))

    i0, f0, g0, o0 = gates(xp_ref[0])
    c1 = i0 * g0
    h1 = o0 * jnp.tanh(c1)
    o_ref[0] = h1

    v1 = xp_ref[1] + jnp.dot(h1.astype(jnp.bfloat16), whh_ref[...],
                             preferred_element_type=jnp.float32)
    i1, f1, g1, o1 = gates(v1)
    c2 = f1 * c1 + i1 * g1
    h2 = o1 * jnp.tanh(c2)
    o_ref[1] = h2


def _lstm(feat, w_ih_t, w_hh_t):
    T, B, D = feat.shape
    H = w_hh_t.shape[0]
    xproj = _mm(feat.reshape(T * B, D), w_ih_t, out_dtype=jnp.float32,
                tn=1024, tk=2048).reshape(T, B, 4 * H)
    return pl.pallas_call(
        functools.partial(_lstm_body, hidden=H),
        out_shape=jax.ShapeDtypeStruct((T, B, H), jnp.float32),
        compiler_params=pltpu.CompilerParams(
            vmem_limit_bytes=50 * 1024 * 1024),
    )(xproj, w_hh_t)


def kernel(x, stem,
           L0_B0_w1, L0_B0_w2, L0_B0_w3, L0_B0_wd,
           L0_B1_w1, L0_B1_w2, L0_B1_w3,
           L0_B2_w1, L0_B2_w2, L0_B2_w3,
           L1_B0_w1, L1_B0_w2, L1_B0_w3, L1_B0_wd,
           L1_B1_w1, L1_B1_w2, L1_B1_w3,
           L1_B2_w1, L1_B2_w2, L1_B2_w3,
           L1_B3_w1, L1_B3_w2, L1_B3_w3,
           L2_B0_w1, L2_B0_w2, L2_B0_w3, L2_B0_wd,
           L2_B1_w1, L2_B1_w2, L2_B1_w3,
           L2_B2_w1, L2_B2_w2, L2_B2_w3,
           L2_B3_w1, L2_B3_w2, L2_B3_w3,
           L2_B4_w1, L2_B4_w2, L2_B4_w3,
           L2_B5_w1, L2_B5_w2, L2_B5_w3,
           L3_B0_w1, L3_B0_w2, L3_B0_w3, L3_B0_wd,
           L3_B1_w1, L3_B1_w2, L3_B1_w3,
           L3_B2_w1, L3_B2_w2, L3_B2_w3,
           w_ih_t, w_hh_t, fc_w_t, fc_b):
    blocks = [
        [(L0_B0_w1, L0_B0_w2, L0_B0_w3, L0_B0_wd),
         (L0_B1_w1, L0_B1_w2, L0_B1_w3, None),
         (L0_B2_w1, L0_B2_w2, L0_B2_w3, None)],
        [(L1_B0_w1, L1_B0_w2, L1_B0_w3, L1_B0_wd),
         (L1_B1_w1, L1_B1_w2, L1_B1_w3, None),
         (L1_B2_w1, L1_B2_w2, L1_B2_w3, None),
         (L1_B3_w1, L1_B3_w2, L1_B3_w3, None)],
        [(L2_B0_w1, L2_B0_w2, L2_B0_w3, L2_B0_wd),
         (L2_B1_w1, L2_B1_w2, L2_B1_w3, None),
         (L2_B2_w1, L2_B2_w2, L2_B2_w3, None),
         (L2_B3_w1, L2_B3_w2, L2_B3_w3, None),
         (L2_B4_w1, L2_B4_w2, L2_B4_w3, None),
         (L2_B5_w1, L2_B5_w2, L2_B5_w3, None)],
        [(L3_B0_w1, L3_B0_w2, L3_B0_w3, L3_B0_wd),
         (L3_B1_w1, L3_B1_w2, L3_B1_w3, None),
         (L3_B2_w1, L3_B2_w2, L3_B2_w3, None)],
    ]

    geoms = [(60, 64, 56), (32, 32, 28), (18, 16, 14), (11, 16, 7)]
    hw = [56, 28, 14, 7]

    v = _maxpool(_stem(x, stem))
    for si, ((_p, _n, stride), stage) in enumerate(zip(_STAGES, blocks)):
        start = 0
        if stride != 1:
            w1, w2, w3, wd = stage[0]
            v = _block(v, w1, w2, w3, wd, stride)
            start = 1
        f = _to_flat(v, geoms[si])
        for w1, w2, w3, wd in stage[start:]:
            f = _fblock(f, w1, w2, w3, wd, geoms[si])
        v = _from_flat(f, geoms[si], hw[si], hw[si])

    fmap_nhwc = v
    pooled = _avgpool(fmap_nhwc)
    feat = pooled.reshape(2, 8, 2048)
    h_all = _lstm(feat, w_ih_t, w_hh_t)
    last = h_all[:, -1, :]
    logits = _mm(last, fc_w_t, bias=fc_b, out_dtype=jnp.float32, tk=2048)
    fmap = fmap_nhwc.transpose(0, 3, 1, 2).astype(jnp.float32)
    return fmap, logits


# speedup vs baseline: 20.6066x; 2.5133x over previous
import functools

import jax
import jax.numpy as jnp
from jax import lax
from jax.experimental import pallas as pl
from jax.experimental.pallas import tpu as pltpu

_STAGES = [(64, 3, 1), (128, 4, 2), (256, 6, 2), (512, 3, 2)]


def _rup(x, m):
    return (x + m - 1) // m * m


def _mm_body(a_ref, b_ref, *rest, has_bias, has_res, act):
    k = pl.program_id(2)

    @pl.when(k == 0)
    def _():
        rest[-1][...] = jnp.zeros_like(rest[-1])

    rest[-1][...] += jnp.dot(a_ref[...], b_ref[...],
                             preferred_element_type=jnp.float32)

    @pl.when(k == pl.num_programs(2) - 1)
    def _():
        r = rest[-1][...]
        if has_bias:
            r = r + rest[0][...]
        if has_res:
            r = r + rest[has_bias][...].astype(jnp.float32)
        if act == "relu":
            r = jnp.maximum(r, 0.0)
        o_ref = rest[has_bias + has_res]
        o_ref[...] = r.astype(o_ref.dtype)


def _mm(a, b, bias=None, res=None, act=None, out_dtype=jnp.bfloat16,
        tm=512, tn=512, tk=1024):
    M, K = a.shape
    _, N = b.shape
    tm = min(tm, _rup(M, 8))
    tn = min(tn, _rup(N, 128))
    tk = min(tk, _rup(K, 128))
    Mp, Np, Kp = _rup(M, tm), _rup(N, tn), _rup(K, tk)

    a = a.astype(jnp.bfloat16)
    b = b.astype(jnp.bfloat16)
    if (Mp, Kp) != (M, K):
        a = jnp.pad(a, ((0, Mp - M), (0, Kp - K)))
    if (Kp, Np) != (K, N):
        b = jnp.pad(b, ((0, Kp - K), (0, Np - N)))

    ops = [a, b]
    specs = [pl.BlockSpec((tm, tk), lambda i, j, k: (i, k)),
             pl.BlockSpec((tk, tn), lambda i, j, k: (k, j))]
    if bias is not None:
        bz = bias.astype(jnp.float32).reshape(1, N)
        if Np != N:
            bz = jnp.pad(bz, ((0, 0), (0, Np - N)))
        ops.append(bz)
        specs.append(pl.BlockSpec((1, tn), lambda i, j, k: (0, j)))
    if res is not None:
        r = res
        if (Mp, Np) != r.shape:
            r = jnp.pad(r, ((0, Mp - M), (0, Np - N)))
        ops.append(r)
        specs.append(pl.BlockSpec((tm, tn), lambda i, j, k: (i, j)))

    out = pl.pallas_call(
        functools.partial(_mm_body, has_bias=bias is not None,
                          has_res=res is not None, act=act),
        out_shape=jax.ShapeDtypeStruct((Mp, Np), out_dtype),
        grid=(Mp // tm, Np // tn, Kp // tk),
        in_specs=specs,
        out_specs=pl.BlockSpec((tm, tn), lambda i, j, k: (i, j)),
        scratch_shapes=[pltpu.VMEM((tm, tn), jnp.float32)],
        compiler_params=pltpu.CompilerParams(
            dimension_semantics=("parallel", "parallel", "arbitrary")),
    )(*ops)
    if (Mp, Np) != (M, N):
        out = out[:M, :N]
    return out


def _stem_body(pe_ref, po_ref, w_ref, o_ref, s_ref):
    s_ref[...] = jnp.zeros_like(s_ref)

    def quad(j, carry):
        for ci in range(3):
            c0 = pe_ref[0, ci, j].astype(jnp.bfloat16)
            c1 = pe_ref[0, ci, j + 1].astype(jnp.bfloat16)
            d0 = po_ref[0, ci, j].astype(jnp.bfloat16)
            d1 = po_ref[0, ci, j + 1].astype(jnp.bfloat16)
            for di in range(4):
                for kh in range(7):
                    r8 = 2 * di + kh
                    ce = c0[r8] if r8 < 8 else c1[r8 - 8]
                    co_ = d0[r8] if r8 < 8 else d1[r8 - 8]
                    for kw in range(7):
                        r = kh * 21 + kw * 3 + ci
                        q = kw // 2 if kw % 2 == 0 else (kw - 1) // 2
                        src = ce if kw % 2 == 0 else co_
                        s_ref[r, di * 128:di * 128 + 112] = src[q:q + 112]
        acc = lax.dot_general(s_ref[...], w_ref[...],
                              (((0,), (0,)), ((), ())),
                              preferred_element_type=jnp.float32)
        y = jnp.maximum(acc, 0.0).astype(o_ref.dtype)
        for di in range(4):
            o_ref[0, pl.ds(4 * j + di, 1)] = y[None, di * 128:di * 128 + 112]
        return carry

    lax.fori_loop(0, 28, quad, 0)


def _stem(x, stem_w):
    xr = x.reshape(16, 3, 224, 224).astype(jnp.bfloat16).astype(jnp.float32)
    xp = jnp.pad(xr, ((0, 0), (0, 0), (3, 13), (3, 3)))
    pe = xp[..., 0::2].reshape(16, 3, 30, 8, 115)
    po = xp[..., 1::2].reshape(16, 3, 30, 8, 115)
    w = jnp.pad(stem_w, ((0, 13), (0, 0)))

    return pl.pallas_call(
        _stem_body,
        out_shape=jax.ShapeDtypeStruct((16, 112, 112, 128), jnp.bfloat16),
        grid=(16,),
        in_specs=[pl.BlockSpec((1, 3, 30, 8, 115), lambda n: (n, 0, 0, 0, 0)),
                  pl.BlockSpec((1, 3, 30, 8, 115), lambda n: (n, 0, 0, 0, 0)),
                  pl.BlockSpec((160, 128), lambda n: (0, 0))],
        out_specs=pl.BlockSpec((1, 112, 112, 128), lambda n: (n, 0, 0, 0)),
        scratch_shapes=[pltpu.VMEM((160, 512), jnp.bfloat16)],
        compiler_params=pltpu.CompilerParams(
            dimension_semantics=("parallel",)),
    )(pe, po, w)


def _maxpool(x):
    return lax.reduce_window(
        x, jnp.asarray(-jnp.inf, dtype=x.dtype), lax.max,
        (1, 3, 3, 1), (1, 2, 2, 1), [(0, 0), (1, 1), (1, 1), (0, 0)])


def _gconv_body(c_ref, w_ref, o_ref, acc_ref):
    t = pl.program_id(2)

    @pl.when(t == 0)
    def _():
        acc_ref[...] = jnp.zeros_like(acc_ref)

    acc_ref[...] += jnp.dot(c_ref[...], w_ref[...],
                            preferred_element_type=jnp.float32)

    @pl.when(t == pl.num_programs(2) - 1)
    def _():
        o_ref[...] = jnp.maximum(acc_ref[...], 0.0).astype(o_ref.dtype)


def _gconv(x, wb, stride):
    N, H, W, C = x.shape
    nb = C // 128
    xp = jnp.pad(x, ((0, 0), (1, 1), (1, 1), (0, 0)))
    Ho = (H + 2 - 3) // stride + 1
    Wo = (W + 2 - 3) // stride + 1
    views = [xp[:, i:i + stride * Ho:stride, j:j + stride * Wo:stride, :]
             for i in range(3) for j in range(3)]
    cols = jnp.concatenate(views, axis=-1).reshape(N * Ho * Wo, 9 * C)
    M = cols.shape[0]
    tm = min(512, _rup(M, 8))
    Mp = _rup(M, tm)
    if Mp != M:
        cols = jnp.pad(cols, ((0, Mp - M), (0, 0)))

    out = pl.pallas_call(
        _gconv_body,
        out_shape=jax.ShapeDtypeStruct((Mp, C), jnp.bfloat16),
        grid=(Mp // tm, nb, 9),
        in_specs=[pl.BlockSpec((tm, 128), lambda i, b, t: (i, t * nb + b)),
                  pl.BlockSpec((128, 128), lambda i, b, t: (b * 9 + t, 0))],
        out_specs=pl.BlockSpec((tm, 128), lambda i, b, t: (i, b)),
        scratch_shapes=[pltpu.VMEM((tm, 128), jnp.float32)],
        compiler_params=pltpu.CompilerParams(
            dimension_semantics=("parallel", "parallel", "arbitrary")),
    )(cols, wb)
    return out[:M].reshape(N, Ho, Wo, C)


def _fblock_body(*refs, geom, nb, has_wd, ksplit):
    Hp, Wp, W = geom
    clen = (Hp - 4) * Wp
    x_ref, w1_ref, wb_ref, w3_ref = refs[:4]
    wd_ref = refs[4] if has_wd else None
    o_ref = refs[4 + has_wd]
    a_ref, gacc_ref = refs[5 + has_wd:7 + has_wd]
    t = pl.program_id(1)

    if ksplit:
        aacc_ref = refs[7 + has_wd]

        @pl.when(t == 0)
        def _():
            aacc_ref[...] = jnp.dot(x_ref[:, :1024], w1_ref[:1024],
                                    preferred_element_type=jnp.float32)

        @pl.when(t == 1)
        def _():
            acc = aacc_ref[...] + jnp.dot(x_ref[:, 1024:], w1_ref[1024:],
                                          preferred_element_type=jnp.float32)
            a_ref[...] = jnp.maximum(acc, 0.0).astype(jnp.bfloat16)
    else:
        @pl.when(t == 0)
        def _():
            a = jnp.dot(x_ref[...], w1_ref[...],
                        preferred_element_type=jnp.float32)
            a_ref[...] = jnp.maximum(a, 0.0).astype(jnp.bfloat16)

    k0 = 2 if ksplit else 1
    for ti in range(9):
        @pl.when(t == k0 + ti)
        def _(ti=ti):
            di, dj = ti // 3, ti % 3
            base = (di + 1) * Wp + dj - 1
            for b in range(nb):
                part = jnp.dot(
                    a_ref[base:base + clen, b * 128:b * 128 + 128],
                    wb_ref[(b * 9 + ti) * 128:(b * 9 + ti) * 128 + 128, :],
                    preferred_element_type=jnp.float32)
                if ti == 0:
                    gacc_ref[:, b * 128:b * 128 + 128] = part
                else:
                    gacc_ref[:, b * 128:b * 128 + 128] += part

    @pl.when(t == k0 + 9)
    def _():
        r = lax.broadcasted_iota(jnp.int32, gacc_ref.shape, 0) % Wp
        colmask = (r >= 1) & (r <= W)
        g = jnp.where(colmask, jnp.maximum(gacc_ref[...], 0.0), 0.0)
        g = g.astype(jnp.bfloat16)
        y = jnp.dot(g, w3_ref[...], preferred_element_type=jnp.float32)
        xc = x_ref[2 * Wp:2 * Wp + clen, :]
        if has_wd:
            res = jnp.dot(xc, wd_ref[...], preferred_element_type=jnp.float32)
            res = res.astype(jnp.bfloat16).astype(jnp.float32)
        else:
            res = xc.astype(jnp.float32)
        outc = o_ref.shape[1]
        o_ref[0:2 * Wp, :] = jnp.zeros((2 * Wp, outc), o_ref.dtype)
        o_ref[2 * Wp:2 * Wp + clen, :] = jnp.maximum(y + res, 0.0).astype(o_ref.dtype)
        o_ref[2 * Wp + clen:, :] = jnp.zeros((2 * Wp, outc), o_ref.dtype)


def _fblock(x, w1, w2, w3, wd, geom):
    Hp, Wp, W = geom
    Rf = Hp * Wp
    Cin = x.shape[1]
    width = w1.shape[1]
    outc = w3.shape[1]
    nb = width // 128
    ksplit = Cin > 1024
    nsteps = (2 if ksplit else 1) + 10
    ops = [x, w1, w2, w3] + ([wd] if wd is not None else [])
    specs = [pl.BlockSpec((Rf, Cin), lambda n, t: (n, 0)),
             pl.BlockSpec(w1.shape, lambda n, t: (0, 0)),
             pl.BlockSpec(w2.shape, lambda n, t: (0, 0)),
             pl.BlockSpec(w3.shape, lambda n, t: (0, 0))]
    if wd is not None:
        specs.append(pl.BlockSpec(wd.shape, lambda n, t: (0, 0)))
    return pl.pallas_call(
        functools.partial(_fblock_body, geom=geom, nb=nb,
                          has_wd=wd is not None, ksplit=ksplit),
        out_shape=jax.ShapeDtypeStruct((16 * Rf, outc), jnp.bfloat16),
        grid=(16, nsteps),
        in_specs=specs,
        out_specs=pl.BlockSpec((Rf, outc), lambda n, t: (n, 0)),
        scratch_shapes=[pltpu.VMEM((Rf, width), jnp.bfloat16),
                        pltpu.VMEM(((Hp - 4) * Wp, width), jnp.float32)]
        + ([pltpu.VMEM((Rf, width), jnp.float32)] if ksplit else []),
        compiler_params=pltpu.CompilerParams(
            dimension_semantics=("parallel", "arbitrary"),
            vmem_limit_bytes=100 * 1024 * 1024),
    )(*ops)


def _fblock2_body(xe_ref, xo_ref, w1_ref, wb_ref, w3_ref, wd_ref, o_ref,
                  ae_ref, ao_ref, gacc_ref, *, gi, go, nb):
    Hpi, Wh = gi
    Hpo, Wpo, Wo = go
    Hout = Hpo - 4
    clen = Hout * Wpo
    L = Hout * 2 * Wh
    t = pl.program_id(1)

    @pl.when(t == 0)
    def _():
        ae = jnp.dot(xe_ref[...], w1_ref[...], preferred_element_type=jnp.float32)
        ae_ref[...] = jnp.maximum(ae, 0.0).astype(jnp.bfloat16)
        ao = jnp.dot(xo_ref[...], w1_ref[...], preferred_element_type=jnp.float32)
        ao_ref[...] = jnp.maximum(ao, 0.0).astype(jnp.bfloat16)

    def slab(src, start, b, cs):
        v = src[start:start + L, b * cs:b * cs + cs]
        v = v.reshape(Hout, 2 * Wh, cs)[:, :Wpo, :]
        return v.reshape(clen, cs)

    for ti in range(9):
        @pl.when(t == 1 + ti)
        def _(ti=ti):
            di, dj = ti // 3, ti % 3
            src = ae_ref if dj != 1 else ao_ref
            off = -1 if dj < 2 else 0
            start = (di + 1) * Wh + off
            for b in range(nb):
                part = jnp.dot(
                    slab(src, start, b, 128),
                    wb_ref[(b * 9 + ti) * 128:(b * 9 + ti) * 128 + 128, :],
                    preferred_element_type=jnp.float32)
                if ti == 0:
                    gacc_ref[:, b * 128:b * 128 + 128] = part
                else:
                    gacc_ref[:, b * 128:b * 128 + 128] += part

    @pl.when(t == 10)
    def _():
        r = lax.broadcasted_iota(jnp.int32, gacc_ref.shape, 0) % Wpo
        colmask = (r >= 1) & (r <= Wo)
        g = jnp.where(colmask, jnp.maximum(gacc_ref[...], 0.0), 0.0)
        g = g.astype(jnp.bfloat16)
        y = jnp.dot(g, w3_ref[...], preferred_element_type=jnp.float32)
        xres = slab(xo_ref, 2 * Wh - 1, 0, xo_ref.shape[1])
        res = jnp.dot(xres, wd_ref[...], preferred_element_type=jnp.float32)
        res = res.astype(jnp.bfloat16).astype(jnp.float32)
        outc = o_ref.shape[1]
        ro = lax.broadcasted_iota(jnp.int32, (clen, outc), 0) % Wpo
        omask = (ro >= 1) & (ro <= Wo)
        out = jnp.where(omask, jnp.maximum(y + res, 0.0), 0.0)
        o_ref[0:2 * Wpo, :] = jnp.zeros((2 * Wpo, outc), o_ref.dtype)
        o_ref[2 * Wpo:2 * Wpo + clen, :] = out.astype(o_ref.dtype)
        o_ref[2 * Wpo + clen:, :] = jnp.zeros((2 * Wpo, outc), o_ref.dtype)


def _fblock2(f, w1, w2, w3, wd, gin, gout):
    Hpi, Wpi, _Wi = gin
    Hpo, Wpo, Wo = gout
    Wh = Wpi // 2
    Cin = f.shape[1]
    width = w1.shape[1]
    outc = w3.shape[1]
    nb = width // 128
    v = f.reshape(16, Hpi, Wpi, Cin)
    xe = v[:, :, 0::2, :].reshape(16 * Hpi * Wh, Cin)
    xo = v[:, :, 1::2, :].reshape(16 * Hpi * Wh, Cin)
    Rfi = Hpi * Wh
    Rfo = Hpo * Wpo
    return pl.pallas_call(
        functools.partial(_fblock2_body, gi=(Hpi, Wh), go=gout, nb=nb),
        out_shape=jax.ShapeDtypeStruct((16 * Rfo, outc), jnp.bfloat16),
        grid=(16, 11),
        in_specs=[pl.BlockSpec((Rfi, Cin), lambda n, t: (n, 0)),
                  pl.BlockSpec((Rfi, Cin), lambda n, t: (n, 0)),
                  pl.BlockSpec(w1.shape, lambda n, t: (0, 0)),
                  pl.BlockSpec(w2.shape, lambda n, t: (0, 0)),
                  pl.BlockSpec(w3.shape, lambda n, t: (0, 0)),
                  pl.BlockSpec(wd.shape, lambda n, t: (0, 0))],
        out_specs=pl.BlockSpec((Rfo, outc), lambda n, t: (n, 0)),
        scratch_shapes=[pltpu.VMEM((Rfi, width), jnp.bfloat16),
                        pltpu.VMEM((Rfi, width), jnp.bfloat16),
                        pltpu.VMEM(((Hpo - 4) * Wpo, width), jnp.float32)],
        compiler_params=pltpu.CompilerParams(
            dimension_semantics=("parallel", "arbitrary"),
            vmem_limit_bytes=100 * 1024 * 1024),
    )(xe, xo, w1, w2, w3, wd)


def _to_flat(v, geom):
    N, H, W, C = v.shape
    Hp, Wp, _ = geom
    vp = jnp.pad(v, ((0, 0), (2, Hp - H - 2), (1, Wp - W - 1), (0, 0)))
    return vp.reshape(N * Hp * Wp, C)


def _from_flat(v, geom, H, W):
    Hp, Wp, _ = geom
    return v.reshape(16, Hp, Wp, -1)[:, 2:2 + H, 1:1 + W, :]


def _block(x, w1, w2, w3, wd, stride):
    N, H, W, C = x.shape
    a = _mm(x.reshape(N * H * W, C), w1, act="relu")
    a = a.reshape(N, H, W, -1)
    g = _gconv(a, w2, stride)
    _, Ho, Wo, Cw = g.shape
    if wd is not None:
        xs = x[:, ::stride, ::stride, :] if stride != 1 else x
        ident = _mm(xs.reshape(N * Ho * Wo, C), wd)
    else:
        ident = x.reshape(N * Ho * Wo, C)
    y = _mm(g.reshape(N * Ho * Wo, Cw), w3, res=ident, act="relu")
    return y.reshape(N, Ho, Wo, -1)


def _pool_body(x_ref, o_ref):
    o_ref[...] = jnp.mean(x_ref[...].astype(jnp.float32), axis=1)


def _avgpool(fmap):
    N, H, W, C = fmap.shape
    return pl.pallas_call(
        _pool_body,
        out_shape=jax.ShapeDtypeStruct((N, C), jnp.float32),
    )(fmap.reshape(N, H * W, C))


def _lstm_body(xp_ref, whh_ref, o_ref, *, hidden):
    H = hidden

    def gates(v):
        return (jax.nn.sigmoid(v[:, 0 * H:1 * H]),
                jax.nn.sigmoid(v[:, 1 * H:2 * H]),
                jnp.tanh(v[:, 2 * H:3 * H]),
                jax.nn.sigmoid(v[:, 3 * H:4 * H]))

    i0, f0, g0, o0 = gates(xp_ref[0])
    c1 = i0 * g0
    h1 = o0 * jnp.tanh(c1)
    o_ref[0] = h1

    v1 = xp_ref[1] + jnp.dot(h1.astype(jnp.bfloat16), whh_ref[...],
                             preferred_element_type=jnp.float32)
    i1, f1, g1, o1 = gates(v1)
    c2 = f1 * c1 + i1 * g1
    h2 = o1 * jnp.tanh(c2)
    o_ref[1] = h2


def _lstm(feat, w_ih_t, w_hh_t):
    T, B, D = feat.shape
    H = w_hh_t.shape[0]
    xproj = _mm(feat.reshape(T * B, D), w_ih_t, out_dtype=jnp.float32,
                tn=1024, tk=2048).reshape(T, B, 4 * H)
    return pl.pallas_call(
        functools.partial(_lstm_body, hidden=H),
        out_shape=jax.ShapeDtypeStruct((T, B, H), jnp.float32),
        compiler_params=pltpu.CompilerParams(
            vmem_limit_bytes=50 * 1024 * 1024),
    )(xproj, w_hh_t)


def kernel(x, stem,
           L0_B0_w1, L0_B0_w2, L0_B0_w3, L0_B0_wd,
           L0_B1_w1, L0_B1_w2, L0_B1_w3,
           L0_B2_w1, L0_B2_w2, L0_B2_w3,
           L1_B0_w1, L1_B0_w2, L1_B0_w3, L1_B0_wd,
           L1_B1_w1, L1_B1_w2, L1_B1_w3,
           L1_B2_w1, L1_B2_w2, L1_B2_w3,
           L1_B3_w1, L1_B3_w2, L1_B3_w3,
           L2_B0_w1, L2_B0_w2, L2_B0_w3, L2_B0_wd,
           L2_B1_w1, L2_B1_w2, L2_B1_w3,
           L2_B2_w1, L2_B2_w2, L2_B2_w3,
           L2_B3_w1, L2_B3_w2, L2_B3_w3,
           L2_B4_w1, L2_B4_w2, L2_B4_w3,
           L2_B5_w1, L2_B5_w2, L2_B5_w3,
           L3_B0_w1, L3_B0_w2, L3_B0_w3, L3_B0_wd,
           L3_B1_w1, L3_B1_w2, L3_B1_w3,
           L3_B2_w1, L3_B2_w2, L3_B2_w3,
           w_ih_t, w_hh_t, fc_w_t, fc_b):
    blocks = [
        [(L0_B0_w1, L0_B0_w2, L0_B0_w3, L0_B0_wd),
         (L0_B1_w1, L0_B1_w2, L0_B1_w3, None),
         (L0_B2_w1, L0_B2_w2, L0_B2_w3, None)],
        [(L1_B0_w1, L1_B0_w2, L1_B0_w3, L1_B0_wd),
         (L1_B1_w1, L1_B1_w2, L1_B1_w3, None),
         (L1_B2_w1, L1_B2_w2, L1_B2_w3, None),
         (L1_B3_w1, L1_B3_w2, L1_B3_w3, None)],
        [(L2_B0_w1, L2_B0_w2, L2_B0_w3, L2_B0_wd),
         (L2_B1_w1, L2_B1_w2, L2_B1_w3, None),
         (L2_B2_w1, L2_B2_w2, L2_B2_w3, None),
         (L2_B3_w1, L2_B3_w2, L2_B3_w3, None),
         (L2_B4_w1, L2_B4_w2, L2_B4_w3, None),
         (L2_B5_w1, L2_B5_w2, L2_B5_w3, None)],
        [(L3_B0_w1, L3_B0_w2, L3_B0_w3, L3_B0_wd),
         (L3_B1_w1, L3_B1_w2, L3_B1_w3, None),
         (L3_B2_w1, L3_B2_w2, L3_B2_w3, None)],
    ]

    geoms = [(60, 64, 56), (32, 32, 28), (18, 16, 14), (11, 16, 7)]
    hw = [56, 28, 14, 7]

    f = _to_flat(_maxpool(_stem(x, stem)), geoms[0])
    for si, ((_p, _n, stride), stage) in enumerate(zip(_STAGES, blocks)):
        start = 0
        if stride != 1:
            w1, w2, w3, wd = stage[0]
            f = _fblock2(f, w1, w2, w3, wd, geoms[si - 1], geoms[si])
            start = 1
        for w1, w2, w3, wd in stage[start:]:
            f = _fblock(f, w1, w2, w3, wd, geoms[si])

    fmap_nhwc = _from_flat(f, geoms[3], 7, 7)
    pooled = _avgpool(fmap_nhwc)
    feat = pooled.reshape(2, 8, 2048)
    h_all = _lstm(feat, w_ih_t, w_hh_t)
    last = h_all[:, -1, :]
    logits = _mm(last, fc_w_t, bias=fc_b, out_dtype=jnp.float32, tk=2048)
    fmap = fmap_nhwc.transpose(0, 3, 1, 2).astype(jnp.float32)
    return fmap, logits
